# Initial kernel scaffold; baseline (speedup 1.0000x reference)
import jax
import jax.numpy as jnp
from jax.experimental import pallas as pl


def kernel(lidar_F, lidar_C, image, py, px, W_in, W_d2, W_d3, W_d4, W_bot, W_u1, W_u2, W_u3, W_fin, b_fin):
    raise NotImplementedError("write your pallas kernel here")


# speedup vs baseline: 2.1751x; 2.1751x over previous
import functools
import jax
import jax.numpy as jnp
from jax import lax
from jax.experimental import pallas as pl
from jax.experimental.pallas import tpu as pltpu
from jax.experimental.pallas import tpu_sc as plsc

G = 64
V = G * G * G
VH = V // 2
NDUMMY = 128
VHP = VH + NDUMMY

NC, NS = 2, 16
PCH = 128
R = 8192
NCHUNK = V // R


def _morton(x, y, z):
    out = jnp.zeros_like(x)
    for b in range(6):
        out = (out
               | (((x >> b) & 1) << (3 * b + 2))
               | (((y >> b) & 1) << (3 * b + 1))
               | (((z >> b) & 1) << (3 * b)))
    return out


def _sc_scatter(feats, idx2, zer, n_chunks_per_tile):
    rows_per_tile = VHP // NS
    mesh = plsc.VectorSubcoreMesh(core_axis_name="c", subcore_axis_name="s")

    @functools.partial(
        pl.kernel,
        mesh=mesh,
        out_type=jax.ShapeDtypeStruct((NC, VHP, 8), jnp.float32),
        scratch_types=[
            pltpu.VMEM_SHARED((VHP, 8), jnp.float32),
            pltpu.VMEM((n_chunks_per_tile * PCH, 8), jnp.float32),
            pltpu.VMEM((n_chunks_per_tile, PCH), jnp.int32),
        ],
        compiler_params=pltpu.CompilerParams(use_tc_tiling_on_sc=False),
    )
    def k(feats_hbm, idx_hbm, zer_hbm, out_hbm, acc_sh, feats_v, idx_v):
        ci = lax.axis_index("c")
        si = lax.axis_index("s")
        pltpu.sync_copy(zer_hbm.at[pl.ds(si * rows_per_tile, rows_per_tile)],
                        acc_sh.at[pl.ds(si * rows_per_tile, rows_per_tile)])
        pltpu.sync_copy(feats_hbm.at[si], feats_v)
        pltpu.sync_copy(idx_hbm.at[ci, si], idx_v)
        plsc.subcore_barrier()

        def body(c, carry):
            pltpu.sync_copy(feats_v.at[pl.ds(c * PCH, PCH)],
                            acc_sh.at[idx_v.at[c]], add=True)
            return carry

        lax.fori_loop(0, n_chunks_per_tile, body, 0)
        plsc.subcore_barrier()
        pltpu.sync_copy(acc_sh.at[pl.ds(si * rows_per_tile, rows_per_tile)],
                        out_hbm.at[ci, pl.ds(si * rows_per_tile, rows_per_tile)])

    return k(feats, idx2, zer)


def _pool8(x):
    n, c = x.shape
    return jnp.max(x.reshape(n // 8, 8, c), axis=1)


def _rep8(x):
    n, c = x.shape
    return jnp.broadcast_to(x[:, None, :], (n, 8, c)).reshape(n * 8, c)


def _unet_body(acc_ref, w_in, w_d2, w_d3, w_d4, w_bot, w_u1a, w_u1b,
               w_u2a, w_u2b, w_u3a, w_u3b, w_fin, b_fin, out_ref):
    relu = lambda x: jnp.maximum(x, 0.0)
    mm = lambda a, b: jnp.dot(a, b, preferred_element_type=jnp.float32)
    acc = acc_ref[0]
    v0 = acc[:, 0:4] / jnp.maximum(acc[:, 4:5], 1.0)
    s1 = relu(mm(v0, w_in[...]))
    s2 = relu(mm(_pool8(s1), w_d2[...]))
    s4 = relu(mm(_pool8(s2), w_d3[...]))
    s8 = relu(mm(_pool8(s4), w_d4[...]))
    bot = _rep8(relu(mm(s8, w_bot[...])))
    u1 = relu(mm(bot, w_u1a[...]) + mm(s4, w_u1b[...]))
    u2 = relu(_rep8(mm(u1, w_u2a[...])) + mm(s2, w_u2b[...]))
    u3 = relu(_rep8(mm(u2, w_u3a[...])) + mm(s1, w_u3b[...]))
    o = mm(u3, w_fin[...]) + b_fin[...]
    nrm = jnp.sqrt(jnp.sum(o * o, axis=1, keepdims=True))
    out_ref[...] = o / jnp.maximum(nrm, 1e-12)


def _unet_call(acc, ws, interpret=False):
    hpc = VH // R
    wspecs = [pl.BlockSpec(w.shape, lambda i, nd=w.ndim: (0,) * nd)
              for w in ws]
    return pl.pallas_call(
        _unet_body,
        grid=(NCHUNK,),
        in_specs=[pl.BlockSpec((1, R, 8), lambda i: (i // hpc, i % hpc, 0))]
        + wspecs,
        out_specs=pl.BlockSpec((R, 32), lambda i: (i, 0)),
        out_shape=jax.ShapeDtypeStruct((V, 32), jnp.float32),
        interpret=interpret,
    )(acc, *ws)


def _sc_gather(vox, vidp, n_chunks_per_tile):
    bpw = n_chunks_per_tile * PCH
    npts = NC * NS * bpw
    mesh = plsc.VectorSubcoreMesh(core_axis_name="c", subcore_axis_name="s")

    @functools.partial(
        pl.kernel,
        mesh=mesh,
        out_type=jax.ShapeDtypeStruct((npts, 32), jnp.float32),
        scratch_types=[
            pltpu.VMEM((n_chunks_per_tile, PCH), jnp.int32),
            pltpu.VMEM((bpw, 32), jnp.float32),
            pltpu.SemaphoreType.DMA,
        ],
        compiler_params=pltpu.CompilerParams(use_tc_tiling_on_sc=False),
    )
    def k(vox_hbm, vid_hbm, out_hbm, idx_v, rows_v, sem):
        ci = lax.axis_index("c")
        si = lax.axis_index("s")
        wid = si * NC + ci
        base = wid * bpw
        pltpu.sync_copy(vid_hbm.at[wid], idx_v)

        def body(c, carry):
            pltpu.async_copy(vox_hbm.at[idx_v.at[c]],
                             rows_v.at[pl.ds(c * PCH, PCH)], sem).wait()
            return carry

        lax.fori_loop(0, n_chunks_per_tile, body, 0)
        pltpu.sync_copy(rows_v, out_hbm.at[pl.ds(base, bpw)])

    return k(vox, vidp)


def kernel(lidar_F, lidar_C, image, py, px, W_in, W_d2, W_d3, W_d4, W_bot,
           W_u1, W_u2, W_u3, W_fin, b_fin):
    n = lidar_F.shape[0]
    c = lidar_C.astype(jnp.int32)
    vid = _morton(c[:, 0], c[:, 1], c[:, 2])

    nsc = -(-n // (NS * PCH))
    npad = NS * nsc * PCH
    feats = jnp.zeros((npad, 8), jnp.float32)
    feats = feats.at[:n, 0:4].set(lidar_F).at[:n, 4].set(1.0)
    feats = feats.reshape(NS, nsc * PCH, 8)
    vid_pad = jnp.full((npad,), -1, jnp.int32).at[:n].set(vid)
    spread = jnp.arange(npad, dtype=jnp.int32) % NDUMMY
    idx2 = []
    for s in range(NC):
        loc = vid_pad - s * VH
        ok = (loc >= 0) & (loc < VH)
        idx2.append(jnp.where(ok, loc, VH + spread))
    idx2 = jnp.stack(idx2).reshape(NC, NS, nsc, PCH)
    zer = jnp.zeros((VHP, 8), jnp.float32)

    acc = _sc_scatter(feats, idx2, zer, nsc)

    ws = (W_in, W_d2, W_d3, W_d4, W_bot,
          W_u1[:256], W_u1[256:], W_u2[:128], W_u2[128:],
          W_u3[:96], W_u3[96:], W_fin, b_fin.reshape(1, 32))
    vox = _unet_call(acc, ws)

    ngc = -(-n // (NC * NS * PCH))
    npad_g = NC * NS * ngc * PCH
    vid_g = jnp.zeros((npad_g,), jnp.int32).at[:n].set(vid)
    vid_g = vid_g.reshape(NC * NS, ngc, PCH)
    out = _sc_gather(vox, vid_g, ngc)
    return out[:n]


# speedup vs baseline: 2.2731x; 1.0451x over previous
import functools
import jax
import jax.numpy as jnp
from jax import lax
from jax.experimental import pallas as pl
from jax.experimental.pallas import tpu as pltpu
from jax.experimental.pallas import tpu_sc as plsc

G = 64
V = G * G * G
VH = V // 2
NDUMMY = 128
VHP = VH + NDUMMY

NC, NS = 2, 16
PCH = 128
R = 8192
NCHUNK = V // R


def _morton(x, y, z):
    out = jnp.zeros_like(x)
    for b in range(6):
        out = (out
               | (((x >> b) & 1) << (3 * b + 2))
               | (((y >> b) & 1) << (3 * b + 1))
               | (((z >> b) & 1) << (3 * b)))
    return out


def _sc_scatter(feats, idx2, zer, n_chunks_per_tile):
    rows_per_tile = VHP // NS
    mesh = plsc.VectorSubcoreMesh(core_axis_name="c", subcore_axis_name="s")

    @functools.partial(
        pl.kernel,
        mesh=mesh,
        out_type=jax.ShapeDtypeStruct((NC, VHP, 8), jnp.float32),
        scratch_types=[
            pltpu.VMEM_SHARED((VHP, 8), jnp.float32),
            pltpu.VMEM((n_chunks_per_tile * PCH, 8), jnp.float32),
            pltpu.VMEM((n_chunks_per_tile, PCH), jnp.int32),
            pltpu.SemaphoreType.DMA,
        ],
        compiler_params=pltpu.CompilerParams(use_tc_tiling_on_sc=False),
    )
    def k(feats_hbm, idx_hbm, zer_hbm, out_hbm, acc_sh, feats_v, idx_v, sem):
        ci = lax.axis_index("c")
        si = lax.axis_index("s")
        pltpu.sync_copy(zer_hbm.at[pl.ds(si * rows_per_tile, rows_per_tile)],
                        acc_sh.at[pl.ds(si * rows_per_tile, rows_per_tile)])
        pltpu.sync_copy(feats_hbm.at[si], feats_v)
        pltpu.sync_copy(idx_hbm.at[ci, si], idx_v)
        plsc.subcore_barrier()

        def body(c, carry):
            pltpu.async_copy(feats_v.at[pl.ds(c * PCH, PCH)],
                             acc_sh.at[idx_v.at[c]], sem, add=True)
            return carry

        lax.fori_loop(0, n_chunks_per_tile, body, 0)
        pltpu.make_async_copy(feats_hbm.at[si], feats_v, sem).wait()
        plsc.subcore_barrier()
        pltpu.sync_copy(acc_sh.at[pl.ds(si * rows_per_tile, rows_per_tile)],
                        out_hbm.at[ci, pl.ds(si * rows_per_tile, rows_per_tile)])

    return k(feats, idx2, zer)


def _pool8(x, scr):
    n, c = x.shape
    scr[...] = x
    s = [scr[pl.ds(k, n // 8, 8), :] for k in range(8)]
    return jnp.fmax(jnp.fmax(jnp.fmax(s[0], s[1]), jnp.fmax(s[2], s[3])),
                    jnp.fmax(jnp.fmax(s[4], s[5]), jnp.fmax(s[6], s[7])))


def _rep8(x):
    n, c = x.shape
    return jnp.broadcast_to(x[:, None, :], (n, 8, c)).reshape(n * 8, c)


def _unet_body(acc_ref, w_in, w_d2, w_d3, w_d4, w_bot, w_u1a, w_u1b,
               w_u2a, w_u2b, w_u3a, w_u3b, w_fin, b_fin, one32_ref, out_ref,
               p1_scr, p2_scr, p3_scr):
    relu = lambda x: jnp.fmax(x, 0.0)
    mm = lambda a, b: jnp.dot(a, b, preferred_element_type=jnp.float32)
    acc = acc_ref[0]
    v0 = acc[:, 0:4] / jnp.fmax(acc[:, 4:5], 1.0)
    s1 = relu(mm(v0, w_in[...]))
    s2 = relu(mm(_pool8(s1, p1_scr), w_d2[...]))
    s4 = relu(mm(_pool8(s2, p2_scr), w_d3[...]))
    s8 = relu(mm(_pool8(s4, p3_scr), w_d4[...]))
    bot = _rep8(relu(mm(s8, w_bot[...])))
    u1 = relu(mm(bot, w_u1a[...]) + mm(s4, w_u1b[...]))
    u2 = relu(_rep8(mm(u1, w_u2a[...])) + mm(s2, w_u2b[...]))
    u3 = relu(_rep8(mm(u2, w_u3a[...])) + mm(s1, w_u3b[...]))
    o = mm(u3, w_fin[...]) + b_fin[...]
    nrm2 = mm(o * o, one32_ref[...])
    out_ref[...] = o * jax.lax.rsqrt(jnp.fmax(nrm2, 1e-24))


def _unet_call(acc, ws, interpret=False):
    hpc = VH // R
    wspecs = [pl.BlockSpec(w.shape, lambda i, nd=w.ndim: (0,) * nd)
              for w in ws]
    return pl.pallas_call(
        _unet_body,
        grid=(NCHUNK,),
        in_specs=[pl.BlockSpec((1, R, 8), lambda i: (i // hpc, i % hpc, 0))]
        + wspecs,
        out_specs=pl.BlockSpec((R, 32), lambda i: (i, 0)),
        out_shape=jax.ShapeDtypeStruct((V, 32), jnp.float32),
        scratch_shapes=[
            pltpu.VMEM((R, 32), jnp.float32),
            pltpu.VMEM((R // 8, 64), jnp.float32),
            pltpu.VMEM((R // 64, 128), jnp.float32),
        ],
        interpret=interpret,
    )(acc, *ws)


def _sc_gather(vox, vidp, n_chunks_per_tile):
    bpw = n_chunks_per_tile * PCH
    npts = NC * NS * bpw
    mesh = plsc.VectorSubcoreMesh(core_axis_name="c", subcore_axis_name="s")

    @functools.partial(
        pl.kernel,
        mesh=mesh,
        out_type=jax.ShapeDtypeStruct((npts, 32), jnp.float32),
        scratch_types=[
            pltpu.VMEM((n_chunks_per_tile, PCH), jnp.int32),
            pltpu.VMEM((bpw, 32), jnp.float32),
            pltpu.SemaphoreType.DMA,
        ],
        compiler_params=pltpu.CompilerParams(use_tc_tiling_on_sc=False),
    )
    def k(vox_hbm, vid_hbm, out_hbm, idx_v, rows_v, sem):
        ci = lax.axis_index("c")
        si = lax.axis_index("s")
        wid = si * NC + ci
        base = wid * bpw
        pltpu.sync_copy(vid_hbm.at[wid], idx_v)

        def body(c, carry):
            pltpu.async_copy(vox_hbm.at[idx_v.at[c]],
                             rows_v.at[pl.ds(c * PCH, PCH)], sem)
            return carry

        lax.fori_loop(0, n_chunks_per_tile, body, 0)
        pltpu.make_async_copy(vox_hbm.at[pl.ds(0, bpw)], rows_v, sem).wait()
        pltpu.sync_copy(rows_v, out_hbm.at[pl.ds(base, bpw)])

    return k(vox, vidp)


def kernel(lidar_F, lidar_C, image, py, px, W_in, W_d2, W_d3, W_d4, W_bot,
           W_u1, W_u2, W_u3, W_fin, b_fin):
    n = lidar_F.shape[0]
    c = lidar_C.astype(jnp.int32)
    vid = _morton(c[:, 0], c[:, 1], c[:, 2])

    nsc = -(-n // (NS * PCH))
    npad = NS * nsc * PCH
    feats = jnp.zeros((npad, 8), jnp.float32)
    feats = feats.at[:n, 0:4].set(lidar_F).at[:n, 4].set(1.0)
    feats = feats.reshape(NS, nsc * PCH, 8)
    vid_pad = jnp.full((npad,), -1, jnp.int32).at[:n].set(vid)
    spread = jnp.arange(npad, dtype=jnp.int32) % NDUMMY
    idx2 = []
    for s in range(NC):
        loc = vid_pad - s * VH
        ok = (loc >= 0) & (loc < VH)
        idx2.append(jnp.where(ok, loc, VH + spread))
    idx2 = jnp.stack(idx2).reshape(NC, NS, nsc, PCH)
    zer = jnp.zeros((VHP, 8), jnp.float32)

    acc = _sc_scatter(feats, idx2, zer, nsc)

    ws = (W_in, W_d2, W_d3, W_d4, W_bot,
          W_u1[:256], W_u1[256:], W_u2[:128], W_u2[128:],
          W_u3[:96], W_u3[96:], W_fin, b_fin.reshape(1, 32),
          jnp.ones((32, 32), jnp.float32))
    vox = _unet_call(acc, ws)

    ngc = -(-n // (NC * NS * PCH))
    npad_g = NC * NS * ngc * PCH
    vid_g = jnp.zeros((npad_g,), jnp.int32).at[:n].set(vid)
    vid_g = vid_g.reshape(NC * NS, ngc, PCH)
    out = _sc_gather(vox, vid_g, ngc)
    return out[:n]


# speedup vs baseline: 2.6304x; 1.1572x over previous
import functools
import jax
import jax.numpy as jnp
from jax import lax
from jax.experimental import pallas as pl
from jax.experimental.pallas import tpu as pltpu
from jax.experimental.pallas import tpu_sc as plsc

G = 64
V = G * G * G
VH = V // 2
NDUMMY = 128
VHP = VH + NDUMMY

NC, NS = 2, 16
PCH = 128
R = 8192
NCHUNK = V // R


def _morton(x, y, z):
    out = jnp.zeros_like(x)
    for b in range(6):
        out = (out
               | (((x >> b) & 1) << (3 * b + 2))
               | (((y >> b) & 1) << (3 * b + 1))
               | (((z >> b) & 1) << (3 * b)))
    return out


def _sc_scatter(feats, idx2, zer, n_chunks_per_tile):
    rows_per_tile = VHP // NS
    mesh = plsc.VectorSubcoreMesh(core_axis_name="c", subcore_axis_name="s")

    @functools.partial(
        pl.kernel,
        mesh=mesh,
        out_type=jax.ShapeDtypeStruct((NC, VHP, 8), jnp.float32),
        scratch_types=[
            pltpu.VMEM_SHARED((VHP, 8), jnp.float32),
            pltpu.VMEM((n_chunks_per_tile * PCH, 8), jnp.float32),
            pltpu.VMEM((n_chunks_per_tile, PCH), jnp.int32),
            pltpu.SemaphoreType.DMA,
        ],
        compiler_params=pltpu.CompilerParams(use_tc_tiling_on_sc=False),
    )
    def k(feats_hbm, idx_hbm, zer_hbm, out_hbm, acc_sh, feats_v, idx_v, sem):
        ci = lax.axis_index("c")
        si = lax.axis_index("s")
        pltpu.sync_copy(zer_hbm.at[pl.ds(si * rows_per_tile, rows_per_tile)],
                        acc_sh.at[pl.ds(si * rows_per_tile, rows_per_tile)])
        pltpu.sync_copy(feats_hbm.at[si], feats_v)
        pltpu.sync_copy(idx_hbm.at[ci, si], idx_v)
        plsc.subcore_barrier()

        def body(c, carry):
            pltpu.async_copy(feats_v.at[pl.ds(c * PCH, PCH)],
                             acc_sh.at[idx_v.at[c]], sem, add=True)
            return carry

        lax.fori_loop(0, n_chunks_per_tile, body, 0)
        pltpu.make_async_copy(feats_hbm.at[si], feats_v, sem).wait()
        plsc.subcore_barrier()
        pltpu.sync_copy(acc_sh.at[pl.ds(si * rows_per_tile, rows_per_tile)],
                        out_hbm.at[ci, pl.ds(si * rows_per_tile, rows_per_tile)])

    return k(feats, idx2, zer)


def _pool8(x, scr):
    n, c = x.shape
    scr[...] = x
    s = [scr[pl.ds(k, n // 8, 8), :] for k in range(8)]
    return jnp.fmax(jnp.fmax(jnp.fmax(s[0], s[1]), jnp.fmax(s[2], s[3])),
                    jnp.fmax(jnp.fmax(s[4], s[5]), jnp.fmax(s[6], s[7])))


def _rep8(x):
    n, c = x.shape
    return jnp.broadcast_to(x[:, None, :], (n, 8, c)).reshape(n * 8, c)


def _unet_body(acc_ref, w_in, w_d2, w_d3, w_d4, w_bot, w_u1a, w_u1b,
               w_u2a, w_u2b, w_u3a, w_u3b, w_fin, b_fin, one32_ref, out_ref,
               p1_scr, p2_scr, p3_scr):
    relu = lambda x: jnp.fmax(x, 0.0)
    mm = lambda a, b: jnp.dot(a, b, preferred_element_type=jnp.float32)
    acc = acc_ref[0]
    v0 = acc[:, 0:4] / jnp.fmax(acc[:, 4:5], 1.0)
    s1 = relu(mm(v0, w_in[...]))
    s2 = relu(mm(_pool8(s1, p1_scr), w_d2[...]))
    s4 = relu(mm(_pool8(s2, p2_scr), w_d3[...]))
    s8 = relu(mm(_pool8(s4, p3_scr), w_d4[...]))
    bot = _rep8(relu(mm(s8, w_bot[...])))
    u1 = relu(mm(bot, w_u1a[...]) + mm(s4, w_u1b[...]))
    u2 = relu(_rep8(mm(u1, w_u2a[...])) + mm(s2, w_u2b[...]))
    u3 = relu(_rep8(mm(u2, w_u3a[...])) + mm(s1, w_u3b[...]))
    o = mm(u3, w_fin[...]) + b_fin[...]
    nrm2 = mm(o * o, one32_ref[...])
    on = o * jax.lax.rsqrt(jnp.fmax(nrm2, 1e-24))
    p1_scr[...] = on
    for k in range(4):
        out_ref[:, pl.ds(32 * k, 32)] = p1_scr[pl.ds(k, R // 4, 4), :]


def _unet_call(acc, ws, interpret=False):
    hpc = VH // R
    wspecs = [pl.BlockSpec(w.shape, lambda i, nd=w.ndim: (0,) * nd)
              for w in ws]
    return pl.pallas_call(
        _unet_body,
        grid=(NCHUNK,),
        in_specs=[pl.BlockSpec((1, R, 8), lambda i: (i // hpc, i % hpc, 0))]
        + wspecs,
        out_specs=pl.BlockSpec((R // 4, 128), lambda i: (i, 0)),
        out_shape=jax.ShapeDtypeStruct((V // 4, 128), jnp.float32),
        scratch_shapes=[
            pltpu.VMEM((R, 32), jnp.float32),
            pltpu.VMEM((R // 8, 64), jnp.float32),
            pltpu.VMEM((R // 64, 128), jnp.float32),
        ],
        interpret=interpret,
    )(acc, *ws)


def _sc_gather(vox, vidp, n_chunks_per_tile, n, quota):
    bpw = n_chunks_per_tile * PCH
    nw = NC * NS
    last = n - (nw - 1) * quota
    mesh = plsc.VectorSubcoreMesh(core_axis_name="c", subcore_axis_name="s")

    @functools.partial(
        pl.kernel,
        mesh=mesh,
        out_type=jax.ShapeDtypeStruct((n, 32), jnp.float32),
        scratch_types=[
            pltpu.VMEM((n_chunks_per_tile, PCH), jnp.int32),
            pltpu.VMEM((bpw, 32), jnp.float32),
            pltpu.SemaphoreType.DMA,
        ],
        compiler_params=pltpu.CompilerParams(use_tc_tiling_on_sc=False),
    )
    def k(vox_hbm, vid_hbm, out_hbm, idx_v, rows_v, sem):
        ci = lax.axis_index("c")
        si = lax.axis_index("s")
        wid = si * NC + ci
        pltpu.sync_copy(vid_hbm.at[wid], idx_v)

        def body(c, carry):
            pltpu.async_copy(vox_hbm.at[idx_v.at[c]],
                             rows_v.at[pl.ds(c * PCH, PCH)], sem)
            return carry

        lax.fori_loop(0, n_chunks_per_tile, body, 0)
        pltpu.make_async_copy(vox_hbm.at[pl.ds(0, bpw)], rows_v, sem).wait()

        @pl.when(wid < nw - 1)
        def _():
            pltpu.sync_copy(rows_v.at[pl.ds(0, quota)],
                            out_hbm.at[pl.ds(wid * quota, quota)])

        @pl.when(wid == nw - 1)
        def _():
            pltpu.sync_copy(rows_v.at[pl.ds(0, last)],
                            out_hbm.at[pl.ds((nw - 1) * quota, last)])

    return k(vox, vidp)


def kernel(lidar_F, lidar_C, image, py, px, W_in, W_d2, W_d3, W_d4, W_bot,
           W_u1, W_u2, W_u3, W_fin, b_fin):
    n = lidar_F.shape[0]
    c = lidar_C.astype(jnp.int32)
    vid = _morton(c[:, 0], c[:, 1], c[:, 2])

    nsc = -(-n // (NS * PCH))
    npad = NS * nsc * PCH
    feats = jnp.concatenate(
        [lidar_F, jnp.ones((n, 1), jnp.float32), jnp.zeros((n, 3), jnp.float32)],
        axis=1)
    feats = jnp.concatenate(
        [feats, jnp.zeros((npad - n, 8), jnp.float32)], axis=0)
    feats = feats.reshape(NS, nsc * PCH, 8)
    vid_pad = jnp.full((npad,), -1, jnp.int32).at[:n].set(vid)
    spread = jnp.arange(npad, dtype=jnp.int32) % NDUMMY
    idx2 = []
    for s in range(NC):
        loc = vid_pad - s * VH
        ok = (loc >= 0) & (loc < VH)
        idx2.append(jnp.where(ok, loc, VH + spread))
    idx2 = jnp.stack(idx2).reshape(NC, NS, nsc, PCH)
    zer = jnp.zeros((VHP, 8), jnp.float32)

    acc = _sc_scatter(feats, idx2, zer, nsc)

    ws = (W_in, W_d2, W_d3, W_d4, W_bot,
          W_u1[:256], W_u1[256:], W_u2[:128], W_u2[128:],
          W_u3[:96], W_u3[96:], W_fin, b_fin.reshape(1, 32),
          jnp.ones((32, 32), jnp.float32))
    vox = _unet_call(acc, ws).reshape(V, 32)

    nw = NC * NS
    quota = ((-(-n // nw)) + 7) // 8 * 8
    ngc = -(-quota // PCH)
    slot = ngc * PCH
    chunks = []
    for w in range(nw):
        lo = w * quota
        hi = min(lo + quota, n)
        chunks.append(jnp.pad(vid[lo:hi], (0, slot - (hi - lo))))
    vid_g = jnp.stack(chunks).reshape(nw, ngc, PCH)
    return _sc_gather(vox, vid_g, ngc, n, quota)


# speedup vs baseline: 2.7895x; 1.0605x over previous
import functools
import jax
import jax.numpy as jnp
from jax import lax
from jax.experimental import pallas as pl
from jax.experimental.pallas import tpu as pltpu
from jax.experimental.pallas import tpu_sc as plsc

G = 64
V = G * G * G
VH = V // 2
NDUMMY = 8192
VHP = VH + NDUMMY

NC, NS = 2, 16
PCH = 128
R = 8192
NCHUNK = V // R


def _morton(x, y, z):
    out = jnp.zeros_like(x)
    for b in range(6):
        out = (out
               | (((x >> b) & 1) << (3 * b + 2))
               | (((y >> b) & 1) << (3 * b + 1))
               | (((z >> b) & 1) << (3 * b)))
    return out


def _sc_scatter(feats, idx2, zer, n_chunks_per_tile):
    rows_per_tile = VHP // NS
    mesh = plsc.VectorSubcoreMesh(core_axis_name="c", subcore_axis_name="s")

    @functools.partial(
        pl.kernel,
        mesh=mesh,
        out_type=jax.ShapeDtypeStruct((NC, VHP, 8), jnp.float32),
        scratch_types=[
            pltpu.VMEM_SHARED((VHP, 8), jnp.float32),
            pltpu.VMEM((n_chunks_per_tile * PCH, 8), jnp.float32),
            pltpu.VMEM((n_chunks_per_tile, PCH), jnp.int32),
            pltpu.SemaphoreType.DMA,
        ],
        compiler_params=pltpu.CompilerParams(use_tc_tiling_on_sc=False),
    )
    def k(feats_hbm, idx_hbm, zer_hbm, out_hbm, acc_sh, feats_v, idx_v, sem):
        ci = lax.axis_index("c")
        si = lax.axis_index("s")
        pltpu.sync_copy(zer_hbm.at[pl.ds(si * rows_per_tile, rows_per_tile)],
                        acc_sh.at[pl.ds(si * rows_per_tile, rows_per_tile)])
        pltpu.sync_copy(feats_hbm.at[si], feats_v)
        pltpu.sync_copy(idx_hbm.at[ci, si], idx_v)
        plsc.subcore_barrier()

        def body(c, carry):
            pltpu.async_copy(feats_v.at[pl.ds(c * PCH, PCH)],
                             acc_sh.at[idx_v.at[c]], sem, add=True)
            return carry

        lax.fori_loop(0, n_chunks_per_tile, body, 0)
        pltpu.make_async_copy(feats_hbm.at[si], feats_v, sem).wait()
        plsc.subcore_barrier()
        pltpu.sync_copy(acc_sh.at[pl.ds(si * rows_per_tile, rows_per_tile)],
                        out_hbm.at[ci, pl.ds(si * rows_per_tile, rows_per_tile)])

    return k(feats, idx2, zer)


def _pool8_read(scr, n):
    s = [scr[pl.ds(k, n // 8, 8), :] for k in range(8)]
    return jnp.fmax(jnp.fmax(jnp.fmax(s[0], s[1]), jnp.fmax(s[2], s[3])),
                    jnp.fmax(jnp.fmax(s[4], s[5]), jnp.fmax(s[6], s[7])))


def _pool8(x, scr):
    scr[...] = x
    return _pool8_read(scr, x.shape[0])


def _rep8(x):
    n, c = x.shape
    return jnp.broadcast_to(x[:, None, :], (n, 8, c)).reshape(n * 8, c)


def _unet_body(acc_ref, w_in, w_d2, w_d3, w_d4, w_bot, w_u1a, w_u1b,
               w_u2a, w_u2b, w_u3a, w_u3b, w_fin, b_fin, one32_ref, out_ref,
               p1_scr, p2_scr, p3_scr):
    relu = lambda x: jnp.fmax(x, 0.0)
    mm = lambda a, b: jnp.dot(a, b, preferred_element_type=jnp.float32)
    acc_pk = acc_ref[...]
    w_in_v = w_in[...]
    for j in range(16):
        sums_j = acc_pk[:, 8 * j:8 * j + 4]
        cnt_j = acc_pk[:, 8 * j + 4:8 * j + 5]
        t_j = mm(sums_j, w_in_v) / jnp.fmax(cnt_j, 1.0)
        p1_scr[pl.ds(j, R // 16, 16), :] = relu(t_j)
    s1 = p1_scr[...]
    s2 = relu(mm(_pool8_read(p1_scr, R), w_d2[...]))
    s4 = relu(mm(_pool8(s2, p2_scr), w_d3[...]))
    s8 = relu(mm(_pool8(s4, p3_scr), w_d4[...]))
    bot = _rep8(relu(mm(s8, w_bot[...])))
    u1 = relu(mm(bot, w_u1a[...]) + mm(s4, w_u1b[...]))
    u2 = relu(_rep8(mm(u1, w_u2a[...])) + mm(s2, w_u2b[...]))
    u3 = relu(_rep8(mm(u2, w_u3a[...])) + mm(s1, w_u3b[...]))
    o = mm(u3, w_fin[...]) + b_fin[...]
    nrm2 = mm(o * o, one32_ref[...])
    on = o * jax.lax.rsqrt(jnp.fmax(nrm2, 1e-24))
    p1_scr[...] = on
    for k in range(4):
        out_ref[:, pl.ds(32 * k, 32)] = p1_scr[pl.ds(k, R // 4, 4), :]


def _unet_call(acc, ws, interpret=False):
    hpc = VH // R
    bph = (VHP // 16) // (R // 16)
    wspecs = [pl.BlockSpec(w.shape, lambda i, nd=w.ndim: (0,) * nd)
              for w in ws]
    return pl.pallas_call(
        _unet_body,
        grid=(NCHUNK,),
        in_specs=[pl.BlockSpec((R // 16, 128),
                               lambda i: (bph * (i // hpc) + i % hpc, 0))]
        + wspecs,
        out_specs=pl.BlockSpec((R // 4, 128), lambda i: (i, 0)),
        out_shape=jax.ShapeDtypeStruct((V // 4, 128), jnp.float32),
        scratch_shapes=[
            pltpu.VMEM((R, 32), jnp.float32),
            pltpu.VMEM((R // 8, 64), jnp.float32),
            pltpu.VMEM((R // 64, 128), jnp.float32),
        ],
        interpret=interpret,
    )(acc, *ws)


def _sc_gather(vox, vidp, n_chunks_per_tile, n, quota):
    bpw = n_chunks_per_tile * PCH
    nw = NC * NS
    last = n - (nw - 1) * quota
    mesh = plsc.VectorSubcoreMesh(core_axis_name="c", subcore_axis_name="s")

    @functools.partial(
        pl.kernel,
        mesh=mesh,
        out_type=jax.ShapeDtypeStruct((n, 32), jnp.float32),
        scratch_types=[
            pltpu.VMEM((n_chunks_per_tile, PCH), jnp.int32),
            pltpu.VMEM((bpw, 32), jnp.float32),
            pltpu.SemaphoreType.DMA,
        ],
        compiler_params=pltpu.CompilerParams(use_tc_tiling_on_sc=False),
    )
    def k(vox_hbm, vid_hbm, out_hbm, idx_v, rows_v, sem):
        ci = lax.axis_index("c")
        si = lax.axis_index("s")
        wid = si * NC + ci
        pltpu.sync_copy(vid_hbm.at[wid], idx_v)

        def body(c, carry):
            pltpu.async_copy(vox_hbm.at[idx_v.at[c]],
                             rows_v.at[pl.ds(c * PCH, PCH)], sem)
            return carry

        lax.fori_loop(0, n_chunks_per_tile, body, 0)
        pltpu.make_async_copy(vox_hbm.at[pl.ds(0, bpw)], rows_v, sem).wait()

        @pl.when(wid < nw - 1)
        def _():
            pltpu.sync_copy(rows_v.at[pl.ds(0, quota)],
                            out_hbm.at[pl.ds(wid * quota, quota)])

        @pl.when(wid == nw - 1)
        def _():
            pltpu.sync_copy(rows_v.at[pl.ds(0, last)],
                            out_hbm.at[pl.ds((nw - 1) * quota, last)])

    return k(vox, vidp)


def kernel(lidar_F, lidar_C, image, py, px, W_in, W_d2, W_d3, W_d4, W_bot,
           W_u1, W_u2, W_u3, W_fin, b_fin):
    n = lidar_F.shape[0]
    c = lidar_C.astype(jnp.int32)
    vid = _morton(c[:, 0], c[:, 1], c[:, 2])

    nsc = -(-n // (NS * PCH))
    npad = NS * nsc * PCH
    feats = jnp.concatenate(
        [lidar_F, jnp.ones((n, 1), jnp.float32), jnp.zeros((n, 3), jnp.float32)],
        axis=1)
    feats = jnp.concatenate(
        [feats, jnp.zeros((npad - n, 8), jnp.float32)], axis=0)
    feats = feats.reshape(NS, nsc * PCH, 8)
    vid_pad = jnp.full((npad,), -1, jnp.int32).at[:n].set(vid)
    spread = jnp.arange(npad, dtype=jnp.int32) % NDUMMY
    idx2 = []
    for s in range(NC):
        loc = vid_pad - s * VH
        ok = (loc >= 0) & (loc < VH)
        idx2.append(jnp.where(ok, loc, VH + spread))
    idx2 = jnp.stack(idx2).reshape(NC, NS, nsc, PCH)
    zer = jnp.zeros((VHP, 8), jnp.float32)

    acc = _sc_scatter(feats, idx2, zer, nsc)

    ws = (W_in, W_d2, W_d3, W_d4, W_bot,
          W_u1[:256], W_u1[256:], W_u2[:128], W_u2[128:],
          W_u3[:96], W_u3[96:], W_fin, b_fin.reshape(1, 32),
          jnp.ones((32, 32), jnp.float32))
    vox = _unet_call(acc.reshape(2 * VHP // 16, 128), ws).reshape(V, 32)

    nw = NC * NS
    ngc = -(-n // (nw * PCH))
    quota = ngc * PCH
    vid_g = jnp.pad(vid, (0, nw * quota - n)).reshape(nw, ngc, PCH)
    return _sc_gather(vox, vid_g, ngc, n, quota)


# speedup vs baseline: 3.3496x; 1.2008x over previous
import functools
import jax
import jax.numpy as jnp
from jax import lax
from jax.experimental import pallas as pl
from jax.experimental.pallas import tpu as pltpu
from jax.experimental.pallas import tpu_sc as plsc

G = 64
V = G * G * G
VH = V // 2
NDUMMY = 8192
VHP = VH + NDUMMY

NC, NS = 2, 16
PCH = 128
R = 8192
NCHUNK = V // R


def _morton(x, y, z):
    out = jnp.zeros_like(x)
    for b in range(6):
        out = (out
               | (((x >> b) & 1) << (3 * b + 2))
               | (((y >> b) & 1) << (3 * b + 1))
               | (((z >> b) & 1) << (3 * b)))
    return out


def _sc_scatter(feats, idx2, zer, n_chunks_per_tile):
    rows_per_tile = VHP // NS
    mesh = plsc.VectorSubcoreMesh(core_axis_name="c", subcore_axis_name="s")

    @functools.partial(
        pl.kernel,
        mesh=mesh,
        out_type=jax.ShapeDtypeStruct((NC, VHP, 8), jnp.float32),
        scratch_types=[
            pltpu.VMEM_SHARED((VHP, 8), jnp.float32),
            pltpu.VMEM((n_chunks_per_tile * PCH, 8), jnp.float32),
            pltpu.VMEM((n_chunks_per_tile, PCH), jnp.int32),
            pltpu.SemaphoreType.DMA,
        ],
        compiler_params=pltpu.CompilerParams(use_tc_tiling_on_sc=False),
    )
    def k(feats_hbm, idx_hbm, zer_hbm, out_hbm, acc_sh, feats_v, idx_v, sem):
        ci = lax.axis_index("c")
        si = lax.axis_index("s")
        pltpu.sync_copy(zer_hbm.at[pl.ds(si * rows_per_tile, rows_per_tile)],
                        acc_sh.at[pl.ds(si * rows_per_tile, rows_per_tile)])
        pltpu.sync_copy(feats_hbm.at[si], feats_v)
        pltpu.sync_copy(idx_hbm.at[ci, si], idx_v)
        plsc.subcore_barrier()

        def body(c, carry):
            pltpu.async_copy(feats_v.at[pl.ds(c * PCH, PCH)],
                             acc_sh.at[idx_v.at[c]], sem, add=True)
            return carry

        lax.fori_loop(0, n_chunks_per_tile, body, 0)
        pltpu.make_async_copy(feats_hbm.at[si], feats_v, sem).wait()
        plsc.subcore_barrier()
        pltpu.sync_copy(acc_sh.at[pl.ds(si * rows_per_tile, rows_per_tile)],
                        out_hbm.at[ci, pl.ds(si * rows_per_tile, rows_per_tile)])

    return k(feats, idx2, zer)


def _pool8_read(scr, n):
    s = [scr[pl.ds(k, n // 8, 8), :] for k in range(8)]
    return jnp.fmax(jnp.fmax(jnp.fmax(s[0], s[1]), jnp.fmax(s[2], s[3])),
                    jnp.fmax(jnp.fmax(s[4], s[5]), jnp.fmax(s[6], s[7])))


def _pool8(x, scr):
    scr[...] = x
    return _pool8_read(scr, x.shape[0])


def _rep8(x):
    n, c = x.shape
    return jnp.broadcast_to(x[:, None, :], (n, 8, c)).reshape(n * 8, c)


def _unet_body(acc_ref, w_in, w_cnt, w_d2, w_d3, w_d4, w_bot, w_u1a, w_u1b,
               w_u2a, w_u2b, w_u3a, w_u3b, w_fin, b_fin, one32_ref, out_ref,
               p1_scr, p2_scr, p3_scr):
    relu = lambda x: jnp.fmax(x, 0.0)
    mm = lambda a, b: jnp.dot(a, b, preferred_element_type=jnp.float32)
    acc_pk = acc_ref[...]
    t_pk = mm(acc_pk, w_in[...])
    c_pk = mm(acc_pk, w_cnt[...])
    s1_pk = relu(t_pk / jnp.fmax(c_pk, 1.0))
    for j in range(16):
        p1_scr[pl.ds(j, R // 16, 16), :] = s1_pk[:, 32 * j:32 * j + 32]
    s1 = p1_scr[...]
    s2 = relu(mm(_pool8_read(p1_scr, R), w_d2[...]))
    s4 = relu(mm(_pool8(s2, p2_scr), w_d3[...]))
    s8 = relu(mm(_pool8(s4, p3_scr), w_d4[...]))
    bot = _rep8(relu(mm(s8, w_bot[...])))
    u1 = relu(mm(bot, w_u1a[...]) + mm(s4, w_u1b[...]))
    u2 = relu(_rep8(mm(u1, w_u2a[...])) + mm(s2, w_u2b[...]))
    u3 = relu(_rep8(mm(u2, w_u3a[...])) + mm(s1, w_u3b[...]))
    o = mm(u3, w_fin[...]) + b_fin[...]
    nrm2 = mm(o * o, one32_ref[...])
    on = o * jax.lax.rsqrt(jnp.fmax(nrm2, 1e-24))
    p1_scr[...] = on
    for k in range(4):
        out_ref[:, pl.ds(32 * k, 32)] = p1_scr[pl.ds(k, R // 4, 4), :]


def _build_ws(W_in, W_d2, W_d3, W_d4, W_bot, W_u1, W_u2, W_u3, W_fin, b_fin):
    eye16 = jnp.eye(16, dtype=jnp.float32)
    w_big = jnp.kron(eye16, jnp.pad(W_in, ((0, 4), (0, 0))))
    a_cnt = jnp.zeros((8, 32), jnp.float32).at[4, :].set(1.0)
    w_cnt = jnp.kron(eye16, a_cnt)
    return (w_big, w_cnt, W_d2, W_d3, W_d4, W_bot,
            W_u1[:256], W_u1[256:], W_u2[:128], W_u2[128:],
            W_u3[:96], W_u3[96:], W_fin, b_fin.reshape(1, 32),
            jnp.ones((32, 32), jnp.float32))


def _unet_call(acc, ws, interpret=False):
    hpc = VH // R
    bph = (VHP // 16) // (R // 16)
    wspecs = [pl.BlockSpec(w.shape, lambda i, nd=w.ndim: (0,) * nd)
              for w in ws]
    return pl.pallas_call(
        _unet_body,
        grid=(NCHUNK,),
        in_specs=[pl.BlockSpec((R // 16, 128),
                               lambda i: (bph * (i // hpc) + i % hpc, 0))]
        + wspecs,
        out_specs=pl.BlockSpec((R // 4, 128), lambda i: (i, 0)),
        out_shape=jax.ShapeDtypeStruct((V // 4, 128), jnp.float32),
        scratch_shapes=[
            pltpu.VMEM((R, 32), jnp.float32),
            pltpu.VMEM((R // 8, 64), jnp.float32),
            pltpu.VMEM((R // 64, 128), jnp.float32),
        ],
        interpret=interpret,
    )(acc, *ws)


def _sc_gather(vox, vidp, n_chunks_per_tile, n, quota):
    bpw = n_chunks_per_tile * PCH
    nw = NC * NS
    last = n - (nw - 1) * quota
    mesh = plsc.VectorSubcoreMesh(core_axis_name="c", subcore_axis_name="s")

    @functools.partial(
        pl.kernel,
        mesh=mesh,
        out_type=jax.ShapeDtypeStruct((n, 32), jnp.float32),
        scratch_types=[
            pltpu.VMEM((n_chunks_per_tile, PCH), jnp.int32),
            pltpu.VMEM((bpw, 32), jnp.float32),
            pltpu.SemaphoreType.DMA,
        ],
        compiler_params=pltpu.CompilerParams(use_tc_tiling_on_sc=False),
    )
    def k(vox_hbm, vid_hbm, out_hbm, idx_v, rows_v, sem):
        ci = lax.axis_index("c")
        si = lax.axis_index("s")
        wid = si * NC + ci
        pltpu.sync_copy(vid_hbm.at[wid], idx_v)

        def body(c, carry):
            pltpu.async_copy(vox_hbm.at[idx_v.at[c]],
                             rows_v.at[pl.ds(c * PCH, PCH)], sem)
            return carry

        lax.fori_loop(0, n_chunks_per_tile, body, 0)
        pltpu.make_async_copy(vox_hbm.at[pl.ds(0, bpw)], rows_v, sem).wait()

        @pl.when(wid < nw - 1)
        def _():
            pltpu.sync_copy(rows_v.at[pl.ds(0, quota)],
                            out_hbm.at[pl.ds(wid * quota, quota)])

        @pl.when(wid == nw - 1)
        def _():
            pltpu.sync_copy(rows_v.at[pl.ds(0, last)],
                            out_hbm.at[pl.ds((nw - 1) * quota, last)])

    return k(vox, vidp)


def kernel(lidar_F, lidar_C, image, py, px, W_in, W_d2, W_d3, W_d4, W_bot,
           W_u1, W_u2, W_u3, W_fin, b_fin):
    n = lidar_F.shape[0]
    c = lidar_C.astype(jnp.int32)
    vid = _morton(c[:, 0], c[:, 1], c[:, 2])

    nsc = -(-n // (NS * PCH))
    npad = NS * nsc * PCH
    featsT = jnp.concatenate(
        [lidar_F.T, jnp.ones((1, n), jnp.float32),
         jnp.zeros((3, n), jnp.float32)], axis=0)
    featsT = jnp.pad(featsT, ((0, 0), (0, npad - n)))
    feats = featsT.T.reshape(NS, nsc * PCH, 8)
    vid_pad = jnp.full((npad,), -1, jnp.int32).at[:n].set(vid)
    spread = jnp.arange(npad, dtype=jnp.int32) % NDUMMY
    idx2 = []
    for s in range(NC):
        loc = vid_pad - s * VH
        ok = (loc >= 0) & (loc < VH)
        idx2.append(jnp.where(ok, loc, VH + spread))
    idx2 = jnp.stack(idx2).reshape(NC, NS, nsc, PCH)
    zer = jnp.zeros((VHP, 8), jnp.float32)

    acc = _sc_scatter(feats, idx2, zer, nsc)

    ws = _build_ws(W_in, W_d2, W_d3, W_d4, W_bot, W_u1, W_u2, W_u3, W_fin,
                   b_fin)
    vox = _unet_call(acc.reshape(2 * VHP // 16, 128), ws).reshape(V, 32)

    nw = NC * NS
    ngc = -(-n // (nw * PCH))
    quota = ngc * PCH
    vid_g = jnp.pad(vid, (0, nw * quota - n)).reshape(nw, ngc, PCH)
    return _sc_gather(vox, vid_g, ngc, n, quota)


# speedup vs baseline: 3.5351x; 1.0554x over previous
import functools
import jax
import jax.numpy as jnp
from jax import lax
from jax.experimental import pallas as pl
from jax.experimental.pallas import tpu as pltpu
from jax.experimental.pallas import tpu_sc as plsc

G = 64
V = G * G * G
VH = V // 2
NDUMMY = R = 16384
VHP = VH + NDUMMY

NC, NS = 2, 16
PCH = 128
NCHUNK = V // R


def _morton(x, y, z):
    out = jnp.zeros_like(x)
    for b in range(6):
        out = (out
               | (((x >> b) & 1) << (3 * b + 2))
               | (((y >> b) & 1) << (3 * b + 1))
               | (((z >> b) & 1) << (3 * b)))
    return out


def _sc_scatter(feats, idx2, zer, n_chunks_per_tile):
    rows_per_tile = VHP // NS
    mesh = plsc.VectorSubcoreMesh(core_axis_name="c", subcore_axis_name="s")

    @functools.partial(
        pl.kernel,
        mesh=mesh,
        out_type=jax.ShapeDtypeStruct((NC, VHP, 8), jnp.float32),
        scratch_types=[
            pltpu.VMEM_SHARED((VHP, 8), jnp.float32),
            pltpu.VMEM((n_chunks_per_tile * PCH, 8), jnp.float32),
            pltpu.VMEM((n_chunks_per_tile, PCH), jnp.int32),
            pltpu.SemaphoreType.DMA,
        ],
        compiler_params=pltpu.CompilerParams(use_tc_tiling_on_sc=False),
    )
    def k(feats_hbm, idx_hbm, zer_hbm, out_hbm, acc_sh, feats_v, idx_v, sem):
        ci = lax.axis_index("c")
        si = lax.axis_index("s")
        pltpu.sync_copy(zer_hbm.at[pl.ds(si * rows_per_tile, rows_per_tile)],
                        acc_sh.at[pl.ds(si * rows_per_tile, rows_per_tile)])
        pltpu.sync_copy(feats_hbm.at[si], feats_v)
        pltpu.sync_copy(idx_hbm.at[ci, si], idx_v)
        plsc.subcore_barrier()

        def body(c, carry):
            pltpu.async_copy(feats_v.at[pl.ds(c * PCH, PCH)],
                             acc_sh.at[idx_v.at[c]], sem, add=True)
            return carry

        lax.fori_loop(0, n_chunks_per_tile, body, 0)
        pltpu.make_async_copy(feats_hbm.at[si], feats_v, sem).wait()
        plsc.subcore_barrier()
        pltpu.sync_copy(acc_sh.at[pl.ds(si * rows_per_tile, rows_per_tile)],
                        out_hbm.at[ci, pl.ds(si * rows_per_tile, rows_per_tile)])

    return k(feats, idx2, zer)


def _pool8_read(scr, n):
    s = [scr[pl.ds(k, n // 8, 8), :] for k in range(8)]
    return jnp.fmax(jnp.fmax(jnp.fmax(s[0], s[1]), jnp.fmax(s[2], s[3])),
                    jnp.fmax(jnp.fmax(s[4], s[5]), jnp.fmax(s[6], s[7])))


def _pool8(x, scr):
    scr[...] = x
    return _pool8_read(scr, x.shape[0])


def _rep8(x):
    n, c = x.shape
    return jnp.broadcast_to(x[:, None, :], (n, 8, c)).reshape(n * 8, c)


def _unet_body(acc_ref, w_in, w_cnt, w_d2, w_d3, w_d4, w_bot, w_u1a, w_u1b,
               w_u2a, w_u2b, w_u3a, w_u3b, w_fin, b_fin, one32_ref, out_ref,
               p1_scr, p2_scr, p3_scr):
    relu = lambda x: jnp.fmax(x, 0.0)
    mm = lambda a, b: jnp.dot(a, b, preferred_element_type=jnp.float32)
    acc_pk = acc_ref[...]
    t_pk = mm(acc_pk, w_in[...])
    c_pk = mm(acc_pk, w_cnt[...])
    s1_pk = relu(t_pk / jnp.fmax(c_pk, 1.0))
    for j in range(16):
        p1_scr[pl.ds(j, R // 16, 16), :] = s1_pk[:, 32 * j:32 * j + 32]
    s1 = p1_scr[...]
    s2 = relu(mm(_pool8_read(p1_scr, R), w_d2[...]))
    s4 = relu(mm(_pool8(s2, p2_scr), w_d3[...]))
    s8 = relu(mm(_pool8(s4, p3_scr), w_d4[...]))
    bot = _rep8(relu(mm(s8, w_bot[...])))
    u1 = relu(mm(bot, w_u1a[...]) + mm(s4, w_u1b[...]))
    u2 = relu(_rep8(mm(u1, w_u2a[...])) + mm(s2, w_u2b[...]))
    u3 = relu(_rep8(mm(u2, w_u3a[...])) + mm(s1, w_u3b[...]))
    o = mm(u3, w_fin[...]) + b_fin[...]
    nrm2 = mm(o * o, one32_ref[...])
    on = o * jax.lax.rsqrt(jnp.fmax(nrm2, 1e-24))
    p1_scr[...] = on
    for k in range(4):
        out_ref[:, pl.ds(32 * k, 32)] = p1_scr[pl.ds(k, R // 4, 4), :]


def _build_ws(W_in, W_d2, W_d3, W_d4, W_bot, W_u1, W_u2, W_u3, W_fin, b_fin):
    eye16 = jnp.eye(16, dtype=jnp.float32)
    w_big = jnp.kron(eye16, jnp.pad(W_in, ((0, 4), (0, 0))))
    a_cnt = jnp.zeros((8, 32), jnp.float32).at[4, :].set(1.0)
    w_cnt = jnp.kron(eye16, a_cnt)
    return (w_big, w_cnt, W_d2, W_d3, W_d4, W_bot,
            W_u1[:256], W_u1[256:], W_u2[:128], W_u2[128:],
            W_u3[:96], W_u3[96:], W_fin, b_fin.reshape(1, 32),
            jnp.ones((32, 32), jnp.float32))


def _unet_call(acc, ws, interpret=False):
    hpc = VH // R
    bph = (VHP // 16) // (R // 16)
    wspecs = [pl.BlockSpec(w.shape, lambda i, nd=w.ndim: (0,) * nd)
              for w in ws]
    return pl.pallas_call(
        _unet_body,
        grid=(NCHUNK,),
        in_specs=[pl.BlockSpec((R // 16, 128),
                               lambda i: (bph * (i // hpc) + i % hpc, 0))]
        + wspecs,
        out_specs=pl.BlockSpec((R // 4, 128), lambda i: (i, 0)),
        out_shape=jax.ShapeDtypeStruct((V // 4, 128), jnp.float32),
        scratch_shapes=[
            pltpu.VMEM((R, 32), jnp.float32),
            pltpu.VMEM((R // 8, 64), jnp.float32),
            pltpu.VMEM((R // 64, 128), jnp.float32),
        ],
        interpret=interpret,
    )(acc, *ws)


def _sc_gather(vox, vidp, n_chunks_per_tile, n, quota):
    bpw = n_chunks_per_tile * PCH
    nw = NC * NS
    last = n - (nw - 1) * quota
    mesh = plsc.VectorSubcoreMesh(core_axis_name="c", subcore_axis_name="s")

    @functools.partial(
        pl.kernel,
        mesh=mesh,
        out_type=jax.ShapeDtypeStruct((n, 32), jnp.float32),
        scratch_types=[
            pltpu.VMEM((n_chunks_per_tile, PCH), jnp.int32),
            pltpu.VMEM((bpw, 32), jnp.float32),
            pltpu.SemaphoreType.DMA,
        ],
        compiler_params=pltpu.CompilerParams(use_tc_tiling_on_sc=False),
    )
    def k(vox_hbm, vid_hbm, out_hbm, idx_v, rows_v, sem):
        ci = lax.axis_index("c")
        si = lax.axis_index("s")
        wid = si * NC + ci
        pltpu.sync_copy(vid_hbm.at[wid], idx_v)

        def body(c, carry):
            pltpu.async_copy(vox_hbm.at[idx_v.at[c]],
                             rows_v.at[pl.ds(c * PCH, PCH)], sem)
            return carry

        lax.fori_loop(0, n_chunks_per_tile, body, 0)
        pltpu.make_async_copy(vox_hbm.at[pl.ds(0, bpw)], rows_v, sem).wait()

        @pl.when(wid < nw - 1)
        def _():
            pltpu.sync_copy(rows_v.at[pl.ds(0, quota)],
                            out_hbm.at[pl.ds(wid * quota, quota)])

        @pl.when(wid == nw - 1)
        def _():
            pltpu.sync_copy(rows_v.at[pl.ds(0, last)],
                            out_hbm.at[pl.ds((nw - 1) * quota, last)])

    return k(vox, vidp)


def kernel(lidar_F, lidar_C, image, py, px, W_in, W_d2, W_d3, W_d4, W_bot,
           W_u1, W_u2, W_u3, W_fin, b_fin):
    n = lidar_F.shape[0]
    c = lidar_C.astype(jnp.int32)
    vid = _morton(c[:, 0], c[:, 1], c[:, 2])

    nsc = -(-n // (NS * PCH))
    npad = NS * nsc * PCH
    featsT = jnp.concatenate(
        [lidar_F.T, jnp.ones((1, n), jnp.float32),
         jnp.zeros((3, n), jnp.float32)], axis=0)
    featsT = jnp.pad(featsT, ((0, 0), (0, npad - n)))
    feats = featsT.T.reshape(NS, nsc * PCH, 8)
    vid_pad = jnp.full((npad,), -1, jnp.int32).at[:n].set(vid)
    spread = jnp.arange(npad, dtype=jnp.int32) % NDUMMY
    idx2 = []
    for s in range(NC):
        loc = vid_pad - s * VH
        ok = (loc >= 0) & (loc < VH)
        idx2.append(jnp.where(ok, loc, VH + spread))
    idx2 = jnp.stack(idx2).reshape(NC, NS, nsc, PCH)
    zer = jnp.zeros((VHP, 8), jnp.float32)

    acc = _sc_scatter(feats, idx2, zer, nsc)

    ws = _build_ws(W_in, W_d2, W_d3, W_d4, W_bot, W_u1, W_u2, W_u3, W_fin,
                   b_fin)
    vox = _unet_call(acc.reshape(2 * VHP // 16, 128), ws).reshape(V, 32)

    nw = NC * NS
    ngc = -(-n // (nw * PCH))
    quota = ngc * PCH
    vid_g = jnp.pad(vid, (0, nw * quota - n)).reshape(nw, ngc, PCH)
    return _sc_gather(vox, vid_g, ngc, n, quota)


# speedup vs baseline: 3.5525x; 1.0049x over previous
import functools
import jax
import jax.numpy as jnp
from jax import lax
from jax.experimental import pallas as pl
from jax.experimental.pallas import tpu as pltpu
from jax.experimental.pallas import tpu_sc as plsc

G = 64
V = G * G * G
VH = V // 2
NDUMMY = 128
R = 32768
VHP = VH + NDUMMY

NC, NS = 2, 16
PCH = 128
NCHUNK = V // R


def _morton(x, y, z):
    out = jnp.zeros_like(x)
    for b in range(6):
        out = (out
               | (((x >> b) & 1) << (3 * b + 2))
               | (((y >> b) & 1) << (3 * b + 1))
               | (((z >> b) & 1) << (3 * b)))
    return out


def _sc_scatter(feats, idx2, zer, n_chunks_per_tile):
    rows_per_tile = VHP // NS
    out_rows_per_tile = VH // NS
    mesh = plsc.VectorSubcoreMesh(core_axis_name="c", subcore_axis_name="s")

    @functools.partial(
        pl.kernel,
        mesh=mesh,
        out_type=jax.ShapeDtypeStruct((NC, VH, 8), jnp.float32),
        scratch_types=[
            pltpu.VMEM_SHARED((VHP, 8), jnp.float32),
            pltpu.VMEM((n_chunks_per_tile * PCH, 8), jnp.float32),
            pltpu.VMEM((n_chunks_per_tile, PCH), jnp.int32),
            pltpu.SemaphoreType.DMA,
        ],
        compiler_params=pltpu.CompilerParams(use_tc_tiling_on_sc=False),
    )
    def k(feats_hbm, idx_hbm, zer_hbm, out_hbm, acc_sh, feats_v, idx_v, sem):
        ci = lax.axis_index("c")
        si = lax.axis_index("s")
        pltpu.sync_copy(zer_hbm.at[pl.ds(si * rows_per_tile, rows_per_tile)],
                        acc_sh.at[pl.ds(si * rows_per_tile, rows_per_tile)])
        pltpu.sync_copy(feats_hbm.at[si], feats_v)
        pltpu.sync_copy(idx_hbm.at[ci, si], idx_v)
        plsc.subcore_barrier()

        def body(c, carry):
            pltpu.async_copy(feats_v.at[pl.ds(c * PCH, PCH)],
                             acc_sh.at[idx_v.at[c]], sem, add=True)
            return carry

        lax.fori_loop(0, n_chunks_per_tile, body, 0)
        pltpu.make_async_copy(feats_hbm.at[si], feats_v, sem).wait()
        plsc.subcore_barrier()
        pltpu.sync_copy(
            acc_sh.at[pl.ds(si * out_rows_per_tile, out_rows_per_tile)],
            out_hbm.at[ci, pl.ds(si * out_rows_per_tile, out_rows_per_tile)])

    return k(feats, idx2, zer)


def _pool8_read(scr, n):
    s = [scr[pl.ds(k, n // 8, 8), :] for k in range(8)]
    return jnp.fmax(jnp.fmax(jnp.fmax(s[0], s[1]), jnp.fmax(s[2], s[3])),
                    jnp.fmax(jnp.fmax(s[4], s[5]), jnp.fmax(s[6], s[7])))


def _pool8(x, scr):
    scr[...] = x
    return _pool8_read(scr, x.shape[0])


def _rep8(x):
    n, c = x.shape
    return jnp.broadcast_to(x[:, None, :], (n, 8, c)).reshape(n * 8, c)


def _unet_body(acc_ref, w_in, w_cnt, w_d2, w_d3, w_d4, w_bot, w_u1a, w_u1b,
               w_u2a, w_u2b, w_u3a, w_u3b, w_fin, b_fin, one32_ref, out_ref,
               p1_scr, p2_scr, p3_scr):
    relu = lambda x: jnp.fmax(x, 0.0)
    mm = lambda a, b: jnp.dot(a, b, preferred_element_type=jnp.float32)
    acc_pk = acc_ref[...]
    t_pk = mm(acc_pk, w_in[...])
    c_pk = mm(acc_pk, w_cnt[...])
    s1_pk = relu(t_pk / jnp.fmax(c_pk, 1.0))
    for j in range(16):
        p1_scr[pl.ds(j, R // 16, 16), :] = s1_pk[:, 32 * j:32 * j + 32]
    s1 = p1_scr[...]
    s2 = relu(mm(_pool8_read(p1_scr, R), w_d2[...]))
    s4 = relu(mm(_pool8(s2, p2_scr), w_d3[...]))
    s8 = relu(mm(_pool8(s4, p3_scr), w_d4[...]))
    bot = _rep8(relu(mm(s8, w_bot[...])))
    u1 = relu(mm(bot, w_u1a[...]) + mm(s4, w_u1b[...]))
    u2 = relu(_rep8(mm(u1, w_u2a[...])) + mm(s2, w_u2b[...]))
    u3 = relu(_rep8(mm(u2, w_u3a[...])) + mm(s1, w_u3b[...]))
    o = mm(u3, w_fin[...]) + b_fin[...]
    nrm2 = mm(o * o, one32_ref[...])
    on = o * jax.lax.rsqrt(jnp.fmax(nrm2, 1e-24))
    p1_scr[...] = on
    for k in range(4):
        out_ref[:, pl.ds(32 * k, 32)] = p1_scr[pl.ds(k, R // 4, 4), :]


def _build_ws(W_in, W_d2, W_d3, W_d4, W_bot, W_u1, W_u2, W_u3, W_fin, b_fin):
    eye16 = jnp.eye(16, dtype=jnp.float32)
    w_big = jnp.kron(eye16, jnp.pad(W_in, ((0, 4), (0, 0))))
    a_cnt = jnp.zeros((8, 32), jnp.float32).at[4, :].set(1.0)
    w_cnt = jnp.kron(eye16, a_cnt)
    return (w_big, w_cnt, W_d2, W_d3, W_d4, W_bot,
            W_u1[:256], W_u1[256:], W_u2[:128], W_u2[128:],
            W_u3[:96], W_u3[96:], W_fin, b_fin.reshape(1, 32),
            jnp.ones((32, 32), jnp.float32))


def _unet_call(acc, ws, interpret=False):
    hpc = VH // R
    bph = (VH // 16) // (R // 16)
    wspecs = [pl.BlockSpec(w.shape, lambda i, nd=w.ndim: (0,) * nd)
              for w in ws]
    return pl.pallas_call(
        _unet_body,
        grid=(NCHUNK,),
        in_specs=[pl.BlockSpec((R // 16, 128),
                               lambda i: (bph * (i // hpc) + i % hpc, 0))]
        + wspecs,
        out_specs=pl.BlockSpec((R // 4, 128), lambda i: (i, 0)),
        out_shape=jax.ShapeDtypeStruct((V // 4, 128), jnp.float32),
        scratch_shapes=[
            pltpu.VMEM((R, 32), jnp.float32),
            pltpu.VMEM((R // 8, 64), jnp.float32),
            pltpu.VMEM((R // 64, 128), jnp.float32),
        ],
        interpret=interpret,
    )(acc, *ws)


def _sc_gather(vox, vidp, n_chunks_per_tile, n, quota):
    bpw = n_chunks_per_tile * PCH
    nw = NC * NS
    last = n - (nw - 1) * quota
    mesh = plsc.VectorSubcoreMesh(core_axis_name="c", subcore_axis_name="s")

    @functools.partial(
        pl.kernel,
        mesh=mesh,
        out_type=jax.ShapeDtypeStruct((n, 32), jnp.float32),
        scratch_types=[
            pltpu.VMEM((n_chunks_per_tile, PCH), jnp.int32),
            pltpu.VMEM((bpw, 32), jnp.float32),
            pltpu.SemaphoreType.DMA,
        ],
        compiler_params=pltpu.CompilerParams(use_tc_tiling_on_sc=False),
    )
    def k(vox_hbm, vid_hbm, out_hbm, idx_v, rows_v, sem):
        ci = lax.axis_index("c")
        si = lax.axis_index("s")
        wid = si * NC + ci
        pltpu.sync_copy(vid_hbm.at[wid], idx_v)

        def body(c, carry):
            pltpu.async_copy(vox_hbm.at[idx_v.at[c]],
                             rows_v.at[pl.ds(c * PCH, PCH)], sem)
            return carry

        lax.fori_loop(0, n_chunks_per_tile, body, 0)
        pltpu.make_async_copy(vox_hbm.at[pl.ds(0, bpw)], rows_v, sem).wait()

        @pl.when(wid < nw - 1)
        def _():
            pltpu.sync_copy(rows_v.at[pl.ds(0, quota)],
                            out_hbm.at[pl.ds(wid * quota, quota)])

        @pl.when(wid == nw - 1)
        def _():
            pltpu.sync_copy(rows_v.at[pl.ds(0, last)],
                            out_hbm.at[pl.ds((nw - 1) * quota, last)])

    return k(vox, vidp)


def kernel(lidar_F, lidar_C, image, py, px, W_in, W_d2, W_d3, W_d4, W_bot,
           W_u1, W_u2, W_u3, W_fin, b_fin):
    n = lidar_F.shape[0]
    c = lidar_C.astype(jnp.int32)
    vid = _morton(c[:, 0], c[:, 1], c[:, 2])

    nsc = -(-n // (NS * PCH))
    npad = NS * nsc * PCH
    featsT = jnp.concatenate(
        [lidar_F.T, jnp.ones((1, n), jnp.float32),
         jnp.zeros((3, n), jnp.float32)], axis=0)
    featsT = jnp.pad(featsT, ((0, 0), (0, npad - n)))
    feats = featsT.T.reshape(NS, nsc * PCH, 8)
    vid_pad = jnp.full((npad,), -1, jnp.int32).at[:n].set(vid)
    spread = jnp.arange(npad, dtype=jnp.int32) % NDUMMY
    idx2 = []
    for s in range(NC):
        loc = vid_pad - s * VH
        ok = (loc >= 0) & (loc < VH)
        idx2.append(jnp.where(ok, loc, VH + spread))
    idx2 = jnp.stack(idx2).reshape(NC, NS, nsc, PCH)
    zer = jnp.zeros((VHP, 8), jnp.float32)

    acc = _sc_scatter(feats, idx2, zer, nsc)

    ws = _build_ws(W_in, W_d2, W_d3, W_d4, W_bot, W_u1, W_u2, W_u3, W_fin,
                   b_fin)
    vox = _unet_call(acc.reshape(2 * VH // 16, 128), ws).reshape(V, 32)

    nw = NC * NS
    ngc = -(-n // (nw * PCH))
    quota = ngc * PCH
    vid_g = jnp.pad(vid, (0, nw * quota - n)).reshape(nw, ngc, PCH)
    return _sc_gather(vox, vid_g, ngc, n, quota)
